# baseline (speedup 1.0000x reference)
import jax
import jax.numpy as jnp
from jax.experimental import pallas as pl
from jax.experimental.pallas import tpu as pltpu

B = 2048
L = 512
D = 512
E = 8
H = 256
KER = 25
PAD = (KER - 1) // 2
HB = B // 2


def _avg_matrix_in_kernel():
    li = jax.lax.broadcasted_iota(jnp.int32, (L, L), 0)
    ji = jax.lax.broadcasted_iota(jnp.int32, (L, L), 1)
    band = (jnp.abs(li - ji) <= PAD).astype(jnp.float32)
    n0 = jnp.clip(PAD + 1 - li, 0, KER).astype(jnp.float32)
    n1 = jnp.clip(li - (L - 2 - PAD), 0, KER).astype(jnp.float32)
    n = jnp.where(ji == 0, n0, jnp.where(ji == L - 1, n1, band))
    return n * (1.0 / KER)


def _gates_transposed(x, w1, w2):
    h = jnp.maximum(jnp.dot(x, w1, preferred_element_type=jnp.float32), 0.0)
    logits = jnp.dot(h, w2, preferred_element_type=jnp.float32)
    lt = jnp.transpose(logits)
    m = jnp.max(lt, axis=0, keepdims=True)
    p = jnp.exp(lt - m)
    probs = p / jnp.sum(p, axis=0, keepdims=True)
    idx = jax.lax.broadcasted_iota(jnp.int32, probs.shape, 0)
    v1 = jnp.max(probs, axis=0, keepdims=True)
    a1 = jnp.min(jnp.where(probs == v1, idx, E), axis=0, keepdims=True)
    masked = jnp.where(idx == a1, -jnp.inf, probs)
    v2 = jnp.max(masked, axis=0, keepdims=True)
    a2 = jnp.min(jnp.where(masked == v2, idx, E), axis=0, keepdims=True)
    denom = v1 + v2 + 1e-6
    return (jnp.where(idx == a1, v1 / denom, 0.0)
            + jnp.where(idx == a2, v2 / denom, 0.0))


def _moe_kernel(x_ref, w1_ref, w2_ref, sw_ref, tw_ref, sb_ref, tb_ref,
                y_ref, loss_ref, x16_ref, g_ref, imp_ref, load_ref):
    s = pl.program_id(0)
    t = pl.program_id(1)

    @pl.when(s == 0)
    def _gate():
        x = x_ref[...]
        gates_t = _gates_transposed(x, w1_ref[...], w2_ref[...])
        blk_imp = jnp.sum(gates_t, axis=1, keepdims=True)
        blk_load = jnp.sum((gates_t > 0).astype(jnp.float32), axis=1,
                           keepdims=True)

        @pl.when(t == 0)
        def _():
            imp_ref[...] = blk_imp
            load_ref[...] = blk_load

        @pl.when(t == 1)
        def _():
            def cv2(v):
                mu = jnp.mean(v)
                var = jnp.sum((v - mu) ** 2) / (E - 1)
                return var / (mu * mu + 1e-10)

            imp = imp_ref[...] + blk_imp
            load = load_ref[...] + blk_load
            loss_ref[...] = jnp.reshape((cv2(imp) + cv2(load)) * 1e-2, (1, 1))

        g = jnp.transpose(gates_t)
        g_ref[pl.ds(t * HB, HB), :] = g
        x16_ref[pl.ds(t * HB, HB), :] = x.astype(jnp.bfloat16)
        bsum = sb_ref[...] + tb_ref[...]
        y_ref[pl.ds(t * HB, HB), :] = jnp.dot(
            g, bsum, preferred_element_type=jnp.float32)

    @pl.when(s >= 1)
    def _fold_apply():
        swe = sw_ref[0]
        diff = (tw_ref[0] - swe).astype(jnp.bfloat16)
        a16 = _avg_matrix_in_kernel().astype(jnp.bfloat16)
        fold = jax.lax.dot_general(a16, diff, (((0,), (1,)), ((), ())),
                                   preferred_element_type=jnp.float32)
        u = (swe.T + fold).astype(jnp.bfloat16)
        e = (s - 1) * 2 + t
        oh = (jax.lax.broadcasted_iota(jnp.int32, (1, E), 1) == e
              ).astype(jnp.float32)
        ge = jnp.sum(g_ref[...] * oh, axis=1, keepdims=True)
        pe = jnp.dot(ge.astype(jnp.bfloat16) * x16_ref[...], u,
                     preferred_element_type=jnp.float32)
        y_ref[...] += pe


def kernel(x_enc, gate_w1, gate_w2, sw, sb, tw, tb):
    mean = x_enc[:, :, 0]

    y, loss = pl.pallas_call(
        _moe_kernel,
        grid=(1 + E // 2, 2),
        in_specs=[
            pl.BlockSpec((HB, L), lambda s, t: (jnp.where(s == 0, t, 1), 0)),
            pl.BlockSpec((L, H), lambda s, t: (0, 0)),
            pl.BlockSpec((H, E), lambda s, t: (0, 0)),
            pl.BlockSpec((1, D, L),
                         lambda s, t: (jnp.clip((s - 1) * 2 + t, 0, E - 1), 0, 0)),
            pl.BlockSpec((1, D, L),
                         lambda s, t: (jnp.clip((s - 1) * 2 + t, 0, E - 1), 0, 0)),
            pl.BlockSpec((E, D), lambda s, t: (0, 0)),
            pl.BlockSpec((E, D), lambda s, t: (0, 0)),
        ],
        out_specs=[
            pl.BlockSpec((B, D), lambda s, t: (0, 0)),
            pl.BlockSpec((1, 1), lambda s, t: (0, 0)),
        ],
        out_shape=[
            jax.ShapeDtypeStruct((B, D), jnp.float32),
            jax.ShapeDtypeStruct((1, 1), jnp.float32),
        ],
        scratch_shapes=[
            pltpu.VMEM((B, L), jnp.bfloat16),
            pltpu.VMEM((B, E), jnp.float32),
            pltpu.VMEM((E, 1), jnp.float32),
            pltpu.VMEM((E, 1), jnp.float32),
        ],
    )(mean, gate_w1, gate_w2, sw, tw, sb, tb)

    return y[:, :, None], loss[0, 0]


# speedup vs baseline: 1.0493x; 1.0493x over previous
import jax
import jax.numpy as jnp
from jax.experimental import pallas as pl
from jax.experimental.pallas import tpu as pltpu

B = 2048
L = 512
D = 512
E = 8
H = 256
KER = 25
PAD = (KER - 1) // 2
EPS = 2
NPAIR = E // EPS


def _avg_matrix_in_kernel():
    li = jax.lax.broadcasted_iota(jnp.int32, (L, L), 0)
    ji = jax.lax.broadcasted_iota(jnp.int32, (L, L), 1)
    band = (jnp.abs(li - ji) <= PAD).astype(jnp.float32)
    n0 = jnp.clip(PAD + 1 - li, 0, KER).astype(jnp.float32)
    n1 = jnp.clip(li - (L - 2 - PAD), 0, KER).astype(jnp.float32)
    n = jnp.where(ji == 0, n0, jnp.where(ji == L - 1, n1, band))
    return n * (1.0 / KER)


def _gates_transposed(x, w1, w2):
    h = jnp.maximum(jnp.dot(x, w1, preferred_element_type=jnp.float32), 0.0)
    logits = jnp.dot(h, w2, preferred_element_type=jnp.float32)
    lt = jnp.transpose(logits)
    m = jnp.max(lt, axis=0, keepdims=True)
    p = jnp.exp(lt - m)
    probs = p / jnp.sum(p, axis=0, keepdims=True)
    idx = jax.lax.broadcasted_iota(jnp.int32, probs.shape, 0)
    v1 = jnp.max(probs, axis=0, keepdims=True)
    a1 = jnp.min(jnp.where(probs == v1, idx, E), axis=0, keepdims=True)
    masked = jnp.where(idx == a1, -jnp.inf, probs)
    v2 = jnp.max(masked, axis=0, keepdims=True)
    a2 = jnp.min(jnp.where(masked == v2, idx, E), axis=0, keepdims=True)
    denom = v1 + v2 + 1e-6
    return (jnp.where(idx == a1, v1 / denom, 0.0)
            + jnp.where(idx == a2, v2 / denom, 0.0))


def _moe_kernel(x_ref, w1_ref, w2_ref, sw_ref, tw_ref, sb_ref, tb_ref,
                y_ref, loss_ref, x16_ref, g_ref):
    s = pl.program_id(0)

    @pl.when(s == 0)
    def _gate():
        x = x_ref[...]
        gates_t = _gates_transposed(x, w1_ref[...], w2_ref[...])

        def cv2(v):
            mu = jnp.mean(v)
            var = jnp.sum((v - mu) ** 2) / (E - 1)
            return var / (mu * mu + 1e-10)

        imp = jnp.sum(gates_t, axis=1, keepdims=True)
        load = jnp.sum((gates_t > 0).astype(jnp.float32), axis=1, keepdims=True)
        loss_ref[...] = jnp.reshape((cv2(imp) + cv2(load)) * 1e-2, (1, 1))

        g = jnp.transpose(gates_t)
        g_ref[...] = g
        x16_ref[...] = x.astype(jnp.bfloat16)
        bsum = sb_ref[...] + tb_ref[...]
        y_ref[...] = jnp.dot(g, bsum, preferred_element_type=jnp.float32)

    @pl.when(s >= 1)
    def _fold_apply():
        a16 = _avg_matrix_in_kernel().astype(jnp.bfloat16)
        xb = x16_ref[...]
        g = g_ref[...]
        total = None
        for k in range(EPS):
            swe = sw_ref[k]
            diff = (tw_ref[k] - swe).astype(jnp.bfloat16)
            fold = jax.lax.dot_general(a16, diff, (((0,), (1,)), ((), ())),
                                       preferred_element_type=jnp.float32)
            u = (swe.T + fold).astype(jnp.bfloat16)
            e = (s - 1) * EPS + k
            oh = (jax.lax.broadcasted_iota(jnp.int32, (1, E), 1) == e
                  ).astype(jnp.float32)
            ge = jnp.sum(g * oh, axis=1, keepdims=True)
            pe = jnp.dot(ge.astype(jnp.bfloat16) * xb, u,
                         preferred_element_type=jnp.float32)
            total = pe if total is None else total + pe
        y_ref[...] += total


def kernel(x_enc, gate_w1, gate_w2, sw, sb, tw, tb):
    mean = x_enc[:, :, 0]

    y, loss = pl.pallas_call(
        _moe_kernel,
        grid=(1 + NPAIR,),
        in_specs=[
            pl.BlockSpec((B, L), lambda s: (0, 0)),
            pl.BlockSpec((L, H), lambda s: (0, 0)),
            pl.BlockSpec((H, E), lambda s: (0, 0)),
            pl.BlockSpec((EPS, D, L), lambda s: (jnp.clip(s - 1, 0, NPAIR - 1), 0, 0)),
            pl.BlockSpec((EPS, D, L), lambda s: (jnp.clip(s - 1, 0, NPAIR - 1), 0, 0)),
            pl.BlockSpec((E, D), lambda s: (0, 0)),
            pl.BlockSpec((E, D), lambda s: (0, 0)),
        ],
        out_specs=[
            pl.BlockSpec((B, D), lambda s: (0, 0)),
            pl.BlockSpec((1, 1), lambda s: (0, 0)),
        ],
        out_shape=[
            jax.ShapeDtypeStruct((B, D), jnp.float32),
            jax.ShapeDtypeStruct((1, 1), jnp.float32),
        ],
        scratch_shapes=[
            pltpu.VMEM((B, L), jnp.bfloat16),
            pltpu.VMEM((B, E), jnp.float32),
        ],
    )(mean, gate_w1, gate_w2, sw, tw, sb, tb)

    return y[:, :, None], loss[0, 0]
